# baseline (speedup 1.0000x reference)
import functools

import jax
import jax.numpy as jnp
from jax import lax
from jax.experimental import pallas as pl
from jax.experimental.pallas import tpu as pltpu

_PAD = 10
_K = 2 * _PAD + 1
_EPS = 1e-8
_BLK = 256


def _take(x, axis, start, length):
    if axis == 0:
        return x[start:start + length, :]
    return x[:, start:start + length]


def _pool(x, axis, op):
    n = x.shape[axis]
    w = x
    for s in (1, 2, 4, 8):
        n -= s
        w = op(_take(w, axis, 0, n), _take(w, axis, s, n))
    n_out = x.shape[axis] - (_K - 1)
    return op(_take(w, axis, 0, n_out), _take(w, axis, 5, n_out))


def _split_limbs(x):
    hi = x.astype(jnp.bfloat16)
    lo = (x - hi.astype(jnp.float32)).astype(jnp.bfloat16)
    return hi, lo


def _band_right():
    i = lax.broadcasted_iota(jnp.int32, (_BLK, _BLK), 0)
    j = lax.broadcasted_iota(jnp.int32, (_BLK, _BLK), 1)
    d = i - j
    m1 = ((d >= 0) & (d <= _K - 1)).astype(jnp.bfloat16)
    i2 = lax.broadcasted_iota(jnp.int32, (_K - 1, _BLK), 0)
    j2 = lax.broadcasted_iota(jnp.int32, (_K - 1, _BLK), 1)
    d2 = _BLK + i2 - j2
    m2 = ((d2 >= 0) & (d2 <= _K - 1)).astype(jnp.bfloat16)
    return m1, m2


def _band_left():
    i = lax.broadcasted_iota(jnp.int32, (_BLK, _BLK), 0)
    p = lax.broadcasted_iota(jnp.int32, (_BLK, _BLK), 1)
    d = p - i
    l1 = ((d >= 0) & (d <= _K - 1)).astype(jnp.bfloat16)
    i2 = lax.broadcasted_iota(jnp.int32, (_BLK, _K - 1), 0)
    t2 = lax.broadcasted_iota(jnp.int32, (_BLK, _K - 1), 1)
    d2 = _BLK + t2 - i2
    l2 = ((d2 >= 0) & (d2 <= _K - 1)).astype(jnp.bfloat16)
    return l1, l2


def _dot(x, y):
    return jax.lax.dot_general(
        x, y, (((1,), (0,)), ((), ())),
        preferred_element_type=jnp.float32)


def _wsum_cols_mxu(x, m1, m2, n_out):
    hi, lo = _split_limbs(x)
    m1p = jnp.concatenate([m1, m1], axis=0)
    m2p = jnp.concatenate([m2, m2], axis=0)
    blocks = []
    for c in range(0, n_out, _BLK):
        main = jnp.concatenate(
            [hi[:, c:c + _BLK], lo[:, c:c + _BLK]], axis=1)
        tail = jnp.concatenate(
            [hi[:, c + _BLK:c + _BLK + _K - 1],
             lo[:, c + _BLK:c + _BLK + _K - 1]], axis=1)
        blocks.append(_dot(main, m1p) + _dot(tail, m2p))
    return jnp.concatenate(blocks, axis=1)


def _wsum_rows_mxu(x, l1, l2, n_out):
    hi, lo = _split_limbs(x)
    l1p = jnp.concatenate([l1, l1], axis=1)
    l2p = jnp.concatenate([l2, l2], axis=1)
    blocks = []
    for r in range(0, n_out, _BLK):
        main = jnp.concatenate(
            [hi[r:r + _BLK, :], lo[r:r + _BLK, :]], axis=0)
        tail = jnp.concatenate(
            [hi[r + _BLK:r + _BLK + _K - 1, :],
             lo[r + _BLK:r + _BLK + _K - 1, :]], axis=0)
        blocks.append(_dot(l1p, main) + _dot(l2p, tail))
    return jnp.concatenate(blocks, axis=0)


def _plane_kernel(x_ref, o_ref, pad_ref, wmn_ref, wmx_ref):
    x = x_ref[0]
    H, W = x.shape
    p2 = 2 * _PAD
    left = jnp.broadcast_to(x[:, :1], (H, p2))
    right = jnp.broadcast_to(x[:, W - 1:], (H, p2))
    xw = jnp.concatenate([left, x, right], axis=1)
    top = jnp.broadcast_to(xw[:1, :], (p2, W + 2 * p2))
    bot = jnp.broadcast_to(xw[H - 1:, :], (p2, W + 2 * p2))
    pad_ref[...] = jnp.concatenate([top, xw, bot], axis=0)

    def _strip(i, carry):
        rows = pl.ds(8 * i, 8)
        s = pad_ref[rows, :]
        wmn_ref[rows, :] = _pool(s, 1, jnp.minimum)
        wmx_ref[rows, :] = _pool(s, 1, jnp.maximum)
        return carry

    jax.lax.fori_loop(0, (H + 2 * p2) // 8, _strip, 0)

    wmn = wmn_ref[...]
    wmx = wmx_ref[...]
    mn = _pool(wmn, 0, jnp.minimum)
    mx = _pool(wmx, 0, jnp.maximum)

    a = 1.0 / (mx - mn + _EPS)
    b = mn * a

    m1, m2 = _band_right()
    l1, l2 = _band_left()
    sa = _wsum_cols_mxu(a, m1, m2, W)
    sb = _wsum_cols_mxu(b, m1, m2, W)
    sa = _wsum_rows_mxu(sa, l1, l2, H)
    sb = _wsum_rows_mxu(sb, l1, l2, H)

    inv_area = 1.0 / (_K * _K)
    o_ref[0] = (x * sa - sb) * inv_area


def kernel(image):
    B, C, H, W = image.shape
    xr = image.reshape(B * C, H, W)
    out = pl.pallas_call(
        _plane_kernel,
        grid=(B * C,),
        in_specs=[pl.BlockSpec((1, H, W), lambda i: (i, 0, 0))],
        out_specs=pl.BlockSpec((1, H, W), lambda i: (i, 0, 0)),
        out_shape=jax.ShapeDtypeStruct((B * C, H, W), image.dtype),
        scratch_shapes=[
            pltpu.VMEM((H + 4 * _PAD, W + 4 * _PAD), jnp.float32),
            pltpu.VMEM((H + 4 * _PAD, W + 2 * _PAD), jnp.float32),
            pltpu.VMEM((H + 4 * _PAD, W + 2 * _PAD), jnp.float32),
        ],
        compiler_params=pltpu.CompilerParams(
            dimension_semantics=("parallel",),
            vmem_limit_bytes=63 * 1024 * 1024,
        ),
        name="smoothed_adaptive_norm",
    )(xr)
    return out.reshape(B, C, H, W)


# speedup vs baseline: 1.3817x; 1.3817x over previous
import functools

import jax
import jax.numpy as jnp
from jax import lax
from jax.experimental import pallas as pl
from jax.experimental.pallas import tpu as pltpu

_PAD = 10
_K = 2 * _PAD + 1
_EPS = 1e-8
_BLK = 256


def _take(x, axis, start, length):
    if axis == 0:
        return x[start:start + length, :]
    return x[:, start:start + length]


def _pool(x, axis, op):
    n = x.shape[axis]
    w = x
    for s in (1, 2, 4, 8):
        n -= s
        w = op(_take(w, axis, 0, n), _take(w, axis, s, n))
    n_out = x.shape[axis] - (_K - 1)
    return op(_take(w, axis, 0, n_out), _take(w, axis, 5, n_out))


def _split_limbs(x):
    hi = x.astype(jnp.bfloat16)
    lo = (x - hi.astype(jnp.float32)).astype(jnp.bfloat16)
    return hi, lo


def _band_right():
    i = lax.broadcasted_iota(jnp.int32, (_BLK, _BLK), 0)
    j = lax.broadcasted_iota(jnp.int32, (_BLK, _BLK), 1)
    d = i - j
    m1 = ((d >= 0) & (d <= _K - 1)).astype(jnp.bfloat16)
    i2 = lax.broadcasted_iota(jnp.int32, (_K - 1, _BLK), 0)
    j2 = lax.broadcasted_iota(jnp.int32, (_K - 1, _BLK), 1)
    d2 = _BLK + i2 - j2
    m2 = ((d2 >= 0) & (d2 <= _K - 1)).astype(jnp.bfloat16)
    return m1, m2


def _band_left():
    i = lax.broadcasted_iota(jnp.int32, (_BLK, _BLK), 0)
    p = lax.broadcasted_iota(jnp.int32, (_BLK, _BLK), 1)
    d = p - i
    l1 = ((d >= 0) & (d <= _K - 1)).astype(jnp.bfloat16)
    i2 = lax.broadcasted_iota(jnp.int32, (_BLK, _K - 1), 0)
    t2 = lax.broadcasted_iota(jnp.int32, (_BLK, _K - 1), 1)
    d2 = _BLK + t2 - i2
    l2 = ((d2 >= 0) & (d2 <= _K - 1)).astype(jnp.bfloat16)
    return l1, l2


def _dot(x, y):
    return jax.lax.dot_general(
        x, y, (((1,), (0,)), ((), ())),
        preferred_element_type=jnp.float32)


def _wsum_cols_mxu(x, m1, m2, n_out):
    hi, lo = _split_limbs(x)
    m1p = jnp.concatenate([m1, m1], axis=0)
    m2p = jnp.concatenate([m2, m2], axis=0)
    blocks = []
    for c in range(0, n_out, _BLK):
        main = jnp.concatenate(
            [hi[:, c:c + _BLK], lo[:, c:c + _BLK]], axis=1)
        tail = jnp.concatenate(
            [hi[:, c + _BLK:c + _BLK + _K - 1],
             lo[:, c + _BLK:c + _BLK + _K - 1]], axis=1)
        blocks.append(_dot(main, m1p) + _dot(tail, m2p))
    return jnp.concatenate(blocks, axis=1)


def _wsum_rows_mxu(x, l1, l2, n_out):
    hi, lo = _split_limbs(x)
    l1p = jnp.concatenate([l1, l1], axis=1)
    l2p = jnp.concatenate([l2, l2], axis=1)
    blocks = []
    for r in range(0, n_out, _BLK):
        main = jnp.concatenate(
            [hi[r:r + _BLK, :], lo[r:r + _BLK, :]], axis=0)
        tail = jnp.concatenate(
            [hi[r + _BLK:r + _BLK + _K - 1, :],
             lo[r + _BLK:r + _BLK + _K - 1, :]], axis=0)
        blocks.append(_dot(l1p, main) + _dot(l2p, tail))
    return jnp.concatenate(blocks, axis=0)


def _plane_kernel(x_ref, o_ref):
    x = x_ref[0]
    H, W = x.shape
    p2 = 2 * _PAD
    left = jnp.broadcast_to(x[:, :1], (H, p2))
    right = jnp.broadcast_to(x[:, W - 1:], (H, p2))
    xw = jnp.concatenate([left, x, right], axis=1)
    top = jnp.broadcast_to(xw[:1, :], (p2, W + 2 * p2))
    bot = jnp.broadcast_to(xw[H - 1:, :], (p2, W + 2 * p2))
    padded = jnp.concatenate([top, xw, bot], axis=0)

    mns, mxs = [], []
    for r in range(0, H + 2 * p2, 8):
        s = padded[r:r + 8, :]
        mns.append(_pool(s, 1, jnp.minimum))
        mxs.append(_pool(s, 1, jnp.maximum))
    wmn = jnp.concatenate(mns, axis=0)
    wmx = jnp.concatenate(mxs, axis=0)
    mn = _pool(wmn, 0, jnp.minimum)
    mx = _pool(wmx, 0, jnp.maximum)

    a = 1.0 / (mx - mn + _EPS)
    b = mn * a

    m1, m2 = _band_right()
    l1, l2 = _band_left()
    sa = _wsum_cols_mxu(a, m1, m2, W)
    sb = _wsum_cols_mxu(b, m1, m2, W)
    sa = _wsum_rows_mxu(sa, l1, l2, H)
    sb = _wsum_rows_mxu(sb, l1, l2, H)

    inv_area = 1.0 / (_K * _K)
    o_ref[0] = (x * sa - sb) * inv_area


def kernel(image):
    B, C, H, W = image.shape
    xr = image.reshape(B * C, H, W)
    out = pl.pallas_call(
        _plane_kernel,
        grid=(B * C,),
        in_specs=[pl.BlockSpec((1, H, W), lambda i: (i, 0, 0))],
        out_specs=pl.BlockSpec((1, H, W), lambda i: (i, 0, 0)),
        out_shape=jax.ShapeDtypeStruct((B * C, H, W), image.dtype),
        compiler_params=pltpu.CompilerParams(
            dimension_semantics=("parallel",),
        ),
        name="smoothed_adaptive_norm",
    )(xr)
    return out.reshape(B, C, H, W)


# speedup vs baseline: 2.8738x; 2.0799x over previous
import functools

import jax
import jax.numpy as jnp
from jax import lax
from jax.experimental import pallas as pl
from jax.experimental.pallas import tpu as pltpu

_PAD = 10
_K = 2 * _PAD + 1
_EPS = 1e-8
_BLK = 256


def _take(x, axis, start, length):
    if axis == 0:
        return x[start:start + length, :]
    return x[:, start:start + length]


def _pool(x, axis, op):
    n = x.shape[axis]
    w = x
    for s in (1, 2, 4, 8):
        n -= s
        w = op(_take(w, axis, 0, n), _take(w, axis, s, n))
    n_out = x.shape[axis] - (_K - 1)
    return op(_take(w, axis, 0, n_out), _take(w, axis, 5, n_out))


def _split_limbs(x):
    hi = x.astype(jnp.bfloat16)
    lo = (x - hi.astype(jnp.float32)).astype(jnp.bfloat16)
    return hi, lo


def _band_right():
    i = lax.broadcasted_iota(jnp.int32, (_BLK, _BLK), 0)
    j = lax.broadcasted_iota(jnp.int32, (_BLK, _BLK), 1)
    d = i - j
    m1 = ((d >= 0) & (d <= _K - 1)).astype(jnp.bfloat16)
    i2 = lax.broadcasted_iota(jnp.int32, (_K - 1, _BLK), 0)
    j2 = lax.broadcasted_iota(jnp.int32, (_K - 1, _BLK), 1)
    d2 = _BLK + i2 - j2
    m2 = ((d2 >= 0) & (d2 <= _K - 1)).astype(jnp.bfloat16)
    return m1, m2


def _band_left():
    i = lax.broadcasted_iota(jnp.int32, (_BLK, _BLK), 0)
    p = lax.broadcasted_iota(jnp.int32, (_BLK, _BLK), 1)
    d = p - i
    l1 = ((d >= 0) & (d <= _K - 1)).astype(jnp.bfloat16)
    i2 = lax.broadcasted_iota(jnp.int32, (_BLK, _K - 1), 0)
    t2 = lax.broadcasted_iota(jnp.int32, (_BLK, _K - 1), 1)
    d2 = _BLK + t2 - i2
    l2 = ((d2 >= 0) & (d2 <= _K - 1)).astype(jnp.bfloat16)
    return l1, l2


def _dot(x, y):
    return jax.lax.dot_general(
        x, y, (((1,), (0,)), ((), ())),
        preferred_element_type=jnp.float32)


def _wsum_cols_mxu(x, m1, m2, n_out):
    hi, lo = _split_limbs(x)
    m1p = jnp.concatenate([m1, m1], axis=0)
    m2p = jnp.concatenate([m2, m2], axis=0)
    blocks = []
    for c in range(0, n_out, _BLK):
        main = jnp.concatenate(
            [hi[:, c:c + _BLK], lo[:, c:c + _BLK]], axis=1)
        tail = jnp.concatenate(
            [hi[:, c + _BLK:c + _BLK + _K - 1],
             lo[:, c + _BLK:c + _BLK + _K - 1]], axis=1)
        blocks.append(_dot(main, m1p) + _dot(tail, m2p))
    return jnp.concatenate(blocks, axis=1)


def _wsum_rows_mxu(x, l1, l2, n_out):
    hi, lo = _split_limbs(x)
    l1p = jnp.concatenate([l1, l1], axis=1)
    l2p = jnp.concatenate([l2, l2], axis=1)
    blocks = []
    for r in range(0, n_out, _BLK):
        main = jnp.concatenate(
            [hi[r:r + _BLK, :], lo[r:r + _BLK, :]], axis=0)
        tail = jnp.concatenate(
            [hi[r + _BLK:r + _BLK + _K - 1, :],
             lo[r + _BLK:r + _BLK + _K - 1, :]], axis=0)
        blocks.append(_dot(l1p, main) + _dot(l2p, tail))
    return jnp.concatenate(blocks, axis=0)


def _plane_kernel(x_ref, o_ref):
    x = x_ref[0]
    H, W = x.shape
    p2 = 2 * _PAD
    left = jnp.broadcast_to(x[:, :1], (H, p2))
    right = jnp.broadcast_to(x[:, W - 1:], (H, p2))
    xw = jnp.concatenate([left, x, right], axis=1)
    top = jnp.broadcast_to(xw[:1, :], (p2, W + 2 * p2))
    bot = jnp.broadcast_to(xw[H - 1:, :], (p2, W + 2 * p2))
    padded = jnp.concatenate([top, xw, bot], axis=0)

    wmn = _pool(padded, 1, jnp.minimum)
    wmx = _pool(padded, 1, jnp.maximum)
    mn = _pool(wmn, 0, jnp.minimum)
    mx = _pool(wmx, 0, jnp.maximum)

    a = 1.0 / (mx - mn + _EPS)
    b = mn * a

    m1, m2 = _band_right()
    l1, l2 = _band_left()
    sa = _wsum_cols_mxu(a, m1, m2, W)
    sb = _wsum_cols_mxu(b, m1, m2, W)
    sa = _wsum_rows_mxu(sa, l1, l2, H)
    sb = _wsum_rows_mxu(sb, l1, l2, H)

    inv_area = 1.0 / (_K * _K)
    o_ref[0] = (x * sa - sb) * inv_area


def kernel(image):
    B, C, H, W = image.shape
    xr = image.reshape(B * C, H, W)
    out = pl.pallas_call(
        _plane_kernel,
        grid=(B * C,),
        in_specs=[pl.BlockSpec((1, H, W), lambda i: (i, 0, 0))],
        out_specs=pl.BlockSpec((1, H, W), lambda i: (i, 0, 0)),
        out_shape=jax.ShapeDtypeStruct((B * C, H, W), image.dtype),
        compiler_params=pltpu.CompilerParams(
            dimension_semantics=("parallel",),
        ),
        name="smoothed_adaptive_norm",
    )(xr)
    return out.reshape(B, C, H, W)
